# baseline (speedup 1.0000x reference)
import functools

import jax
import jax.numpy as jnp
from jax.experimental import pallas as pl
from jax.experimental.pallas import tpu as pltpu

_N = 10000
_K = 16
_BQ = 400


def _knn_body(x_ref, posT_ref, nbr_ref, feats_ref, *, bq, npad, k):
    i = pl.program_id(0)
    xb = x_ref[...]
    posT = posT_ref[...]
    sq = jnp.sum(posT * posT, axis=0, keepdims=True)

    cross = jnp.zeros((bq, npad), dtype=jnp.float32)
    qsq = jnp.zeros((bq, 1), dtype=jnp.float32)
    for c in range(3):
        qc = xb[:, c:c + 1]
        qb = qc.astype(jnp.bfloat16).astype(jnp.float32)
        pb = posT[c:c + 1, :].astype(jnp.bfloat16).astype(jnp.float32)
        cross = cross + qb * pb
        qsq = qsq + qc * qc
    d = qsq - 2.0 * cross + sq

    col = jax.lax.broadcasted_iota(jnp.int32, (bq, npad), 1)
    rowid = jax.lax.broadcasted_iota(jnp.int32, (bq, 1), 0) + i * bq
    d = jnp.where(col == rowid, jnp.inf, d)

    idx_cols = []
    for _ in range(k):
        m = jnp.min(d, axis=1, keepdims=True)
        idx = jnp.min(jnp.where(d == m, col, npad), axis=1,
                      keepdims=True).astype(jnp.int32)
        idx_cols.append(idx)
        d = jnp.where(col == idx, jnp.inf, d)
    nbr_ref[...] = jnp.concatenate(idx_cols, axis=1)

    feats_ref[...] = jnp.concatenate([xb[:, 0:5], xb[:, 8:9]], axis=1)


def kernel(x, cell_ids):
    n = x.shape[0]
    npad = ((n + 127) // 128) * 128
    posT = x[:, :3].T
    posT = jnp.pad(posT, ((0, 0), (0, npad - n)), constant_values=1e9)
    grid = n // _BQ
    nbr, feats = pl.pallas_call(
        functools.partial(_knn_body, bq=_BQ, npad=npad, k=_K),
        grid=(grid,),
        in_specs=[
            pl.BlockSpec((_BQ, 9), lambda i: (i, 0)),
            pl.BlockSpec((3, npad), lambda i: (0, 0)),
        ],
        out_specs=[
            pl.BlockSpec((_BQ, _K), lambda i: (i, 0)),
            pl.BlockSpec((_BQ, 6), lambda i: (i, 0)),
        ],
        out_shape=[
            jax.ShapeDtypeStruct((n, _K), jnp.int32),
            jax.ShapeDtypeStruct((n, 6), jnp.float32),
        ],
        compiler_params=pltpu.CompilerParams(
            dimension_semantics=("parallel",)),
    )(x, posT)
    src = nbr.reshape(-1)
    dst = jnp.repeat(jnp.arange(n, dtype=jnp.int32), _K)
    edge_index = jnp.stack([src, dst], axis=0)
    return feats, edge_index, cell_ids


# speedup vs baseline: 1.0920x; 1.0920x over previous
import functools

import jax
import jax.numpy as jnp
from jax.experimental import pallas as pl
from jax.experimental.pallas import tpu as pltpu

_N = 10000
_K = 16
_BQ = 1000


def _knn_body(x_ref, posT_ref, nbr_ref, feats_ref, *, bq, npad, k):
    i = pl.program_id(0)
    xb = x_ref[...]
    posT = posT_ref[...]
    sq = jnp.sum(posT * posT, axis=0, keepdims=True)

    cross = jnp.zeros((bq, npad), dtype=jnp.float32)
    qsq = jnp.zeros((bq, 1), dtype=jnp.float32)
    for c in range(3):
        qc = xb[:, c:c + 1]
        qb = qc.astype(jnp.bfloat16).astype(jnp.float32)
        pb = posT[c:c + 1, :].astype(jnp.bfloat16).astype(jnp.float32)
        cross = cross + qb * pb
        qsq = qsq + qc * qc
    d = qsq - 2.0 * cross + sq

    col = jax.lax.broadcasted_iota(jnp.int32, (bq, npad), 1)
    rowid = jax.lax.broadcasted_iota(jnp.int32, (bq, 1), 0) + i * bq
    d = jnp.where(col == rowid, jnp.inf, d)

    idx_cols = []
    for _ in range(k):
        m = jnp.min(d, axis=1, keepdims=True)
        idx = jnp.min(jnp.where(d == m, col, npad), axis=1,
                      keepdims=True).astype(jnp.int32)
        idx_cols.append(idx)
        d = jnp.where(col == idx, jnp.inf, d)
    nbr_ref[...] = jnp.concatenate(idx_cols, axis=1)

    feats_ref[...] = jnp.concatenate([xb[:, 0:5], xb[:, 8:9]], axis=1)


def kernel(x, cell_ids):
    n = x.shape[0]
    npad = ((n + 127) // 128) * 128
    posT = x[:, :3].T
    posT = jnp.pad(posT, ((0, 0), (0, npad - n)), constant_values=1e9)
    grid = n // _BQ
    nbr, feats = pl.pallas_call(
        functools.partial(_knn_body, bq=_BQ, npad=npad, k=_K),
        grid=(grid,),
        in_specs=[
            pl.BlockSpec((_BQ, 9), lambda i: (i, 0)),
            pl.BlockSpec((3, npad), lambda i: (0, 0)),
        ],
        out_specs=[
            pl.BlockSpec((_BQ, _K), lambda i: (i, 0)),
            pl.BlockSpec((_BQ, 6), lambda i: (i, 0)),
        ],
        out_shape=[
            jax.ShapeDtypeStruct((n, _K), jnp.int32),
            jax.ShapeDtypeStruct((n, 6), jnp.float32),
        ],
        compiler_params=pltpu.CompilerParams(
            dimension_semantics=("parallel",)),
    )(x, posT)
    src = nbr.reshape(-1)
    dst = jnp.repeat(jnp.arange(n, dtype=jnp.int32), _K)
    edge_index = jnp.stack([src, dst], axis=0)
    return feats, edge_index, cell_ids
